# baseline (speedup 1.0000x reference)
import functools

import jax
import jax.numpy as jnp
from jax.experimental import pallas as pl

B = 256
L = 96
K = 6
D = 128
ATOM_OUT = 40
BOND_OUT = 10
NEG = -9e8


def _elu(x):
    return jnp.where(x > 0, x, jnp.exp(jnp.minimum(x, 0.0)) - 1.0)


def _gru_block(x, h, wih, whh, bih, bhh):
    gi = jnp.dot(x, wih, preferred_element_type=jnp.float32) + bih
    gh = jnp.dot(h, whh, preferred_element_type=jnp.float32) + bhh
    r = jax.nn.sigmoid(gi[:, :D] + gh[:, :D])
    z = jax.nn.sigmoid(gi[:, D:2 * D] + gh[:, D:2 * D])
    n = jnp.tanh(gi[:, 2 * D:] + r * gh[:, 2 * D:])
    return (1.0 - z) * n + z * h


def _seg_softmax(x, io, lo, hi):
    m = (io >= lo) & (io < hi)
    xs = jnp.where(m, x, NEG)
    mx = jnp.max(xs, axis=-1, keepdims=True)
    e = jnp.exp(xs - mx) * m.astype(jnp.float32)
    return e / jnp.sum(e, axis=-1, keepdims=True)


def _grn_kernel(idx_ref, am_ref, mf_ref, af_ref,
                wm1_ref, wm2_ref, bma_ref, wmat_ref, bmat_ref,
                mwih_ref, mwhh_ref, mbih_ref, mbhh_ref,
                w1c_ref, w2c_ref, bal_ref, wat_ref, batt_ref,
                gwih_ref, gwhh_ref, gbih_ref, gbhh_ref,
                wafc_ref, bafc_ref, wb1_ref, wb2_ref, bb_ref,
                out_ref):
    idx = idx_ref[0]
    am = am_ref[0]
    mf = mf_ref[0]
    af = af_ref[0]

    mfh = jnp.dot(mf, wm1_ref[...], preferred_element_type=jnp.float32)
    afh = jnp.dot(af, wm2_ref[...], preferred_element_type=jnp.float32)
    v = jax.nn.leaky_relu(mfh + afh + bma_ref[...])
    msm = jnp.where(am == 0.0, NEG, 0.0)
    v = (v + msm) * am
    giT = mf * af
    ctx = _elu(
        jnp.dot(v * af, wmat_ref[...], preferred_element_type=jnp.float32)
        + bmat_ref[...])
    act = jax.nn.relu(_gru_block(ctx, giT, mwih_ref[...], mwhh_ref[...],
                                 mbih_ref[...], mbhh_ref[...]))

    jio = jax.lax.broadcasted_iota(jnp.int32, (L, L), 1)
    mk = [(idx[:, k:k + 1] == jio).astype(jnp.float32) for k in range(K)]
    amask = jnp.where(idx != L - 1, 1.0, 0.0)
    smask = jnp.where(idx == L - 1, NEG, 0.0)

    for d in range(2):
        s_self = jnp.dot(act, w1c_ref[d], preferred_element_type=jnp.float32)
        p_col = jnp.dot(act, w2c_ref[d], preferred_element_type=jnp.float32)
        p_g = jnp.concatenate(
            [jnp.dot(mk[k], p_col, preferred_element_type=jnp.float32)
             for k in range(K)], axis=-1)
        score = jax.nn.leaky_relu(s_self + p_g + bal_ref[d]) + smask
        mx = jnp.max(score, axis=1, keepdims=True)
        e = jnp.exp(score - mx)
        attn = e / jnp.sum(e, axis=1, keepdims=True) * amask
        s_mat = attn[:, 0:1] * mk[0]
        for k in range(1, K):
            s_mat = s_mat + attn[:, k:k + 1] * mk[k]
        ctxw = jnp.dot(s_mat, act, preferred_element_type=jnp.float32)
        asum = jnp.sum(attn, axis=1, keepdims=True)
        ctx2 = _elu(
            jnp.dot(ctxw, wat_ref[d], preferred_element_type=jnp.float32)
            + asum * batt_ref[d])
        act = jax.nn.relu(_gru_block(ctx2, act, gwih_ref[d], gwhh_ref[d],
                                     gbih_ref[d], gbhh_ref[d]))

    atom_out = (jnp.dot(act, wafc_ref[...], preferred_element_type=jnp.float32)
                + bafc_ref[...])
    io40 = jax.lax.broadcasted_iota(jnp.int32, (L, ATOM_OUT), 1)
    a = (_seg_softmax(atom_out, io40, 0, 16)
         + _seg_softmax(atom_out, io40, 16, 22)
         + _seg_softmax(atom_out, io40, 24, 30)
         + _seg_softmax(atom_out, io40, 31, 36)
         + _seg_softmax(atom_out, io40, 37, 39))
    a = a + jnp.where(io40 == 24, jax.nn.relu(atom_out), 0.0)
    a = a + jnp.where(io40 == 30, jax.nn.sigmoid(atom_out), 0.0)
    a = a + jnp.where(io40 == 36, jax.nn.sigmoid(atom_out), 0.0)

    r_self = jnp.dot(act, wb1_ref[...], preferred_element_type=jnp.float32)
    q = jnp.dot(act, wb2_ref[...], preferred_element_type=jnp.float32)
    io10 = jax.lax.broadcasted_iota(jnp.int32, (L, BOND_OUT), 1)
    pieces = [a]
    for k in range(K):
        qg = jnp.dot(mk[k], q, preferred_element_type=jnp.float32)
        bo = r_self + qg + bb_ref[...]
        pieces.append(_seg_softmax(bo, io10, 0, 4)
                      + _seg_softmax(bo, io10, 6, 10))
    out_ref[0] = jnp.concatenate(pieces, axis=-1)


@jax.jit
def kernel(atom_list, bond_list, atom_degree_list, bond_degree_list, atom_mask,
           mol_feature, activated_features, W_atom_fc, b_atom_fc, W_bond_fc,
           b_bond_fc, gru_W_ih, gru_W_hh, gru_b_ih, gru_b_hh, W_align, b_align,
           W_attend, b_attend, mol_gru_W_ih, mol_gru_W_hh, mol_gru_b_ih,
           mol_gru_b_hh, W_mol_align, b_mol_align, W_mol_attend, b_mol_attend):
    del atom_list, bond_list, bond_degree_list

    idx = atom_degree_list.astype(jnp.int32)
    am = atom_mask.reshape(B, L, 1)
    mf3 = mol_feature.reshape(B, 1, D)

    wm1 = W_mol_align[:, :D].T
    wm2 = W_mol_align[:, D:].T
    bma = b_mol_align[None, :]
    wmat = W_mol_attend.T
    bmat = b_mol_attend[None, :]
    mwih = mol_gru_W_ih.T
    mwhh = mol_gru_W_hh.T
    mbih = mol_gru_b_ih[None, :]
    mbhh = mol_gru_b_hh[None, :]
    w1c = jnp.stack([W_align[0, :, :D].T, W_align[1, :, :D].T])
    w2c = jnp.stack([W_align[0, :, D:].T, W_align[1, :, D:].T])
    bal = b_align[:2].reshape(2, 1, 1)
    wat = jnp.stack([W_attend[0].T, W_attend[1].T])
    batt = b_attend[:2].reshape(2, 1, D)
    gwih = jnp.stack([gru_W_ih[0].T, gru_W_ih[1].T])
    gwhh = jnp.stack([gru_W_hh[0].T, gru_W_hh[1].T])
    gbih = gru_b_ih[:2].reshape(2, 1, 3 * D)
    gbhh = gru_b_hh[:2].reshape(2, 1, 3 * D)
    wafc = W_atom_fc.T
    bafc = b_atom_fc[None, :]
    wb1 = W_bond_fc[:, :D].T
    wb2 = W_bond_fc[:, D:].T
    bb = b_bond_fc[None, :]

    full = lambda shape: pl.BlockSpec(shape, lambda i: (0,) * len(shape))
    grid_spec = pl.GridSpec(
        grid=(B,),
        in_specs=[
            pl.BlockSpec((1, L, K), lambda i: (i, 0, 0)),
            pl.BlockSpec((1, L, 1), lambda i: (i, 0, 0)),
            pl.BlockSpec((1, 1, D), lambda i: (i, 0, 0)),
            pl.BlockSpec((1, L, D), lambda i: (i, 0, 0)),
            full((D, D)), full((D, D)), full((1, D)), full((D, D)),
            full((1, D)), full((D, 3 * D)), full((D, 3 * D)),
            full((1, 3 * D)), full((1, 3 * D)),
            full((2, D, 1)), full((2, D, 1)), full((2, 1, 1)),
            full((2, D, D)), full((2, 1, D)),
            full((2, D, 3 * D)), full((2, D, 3 * D)),
            full((2, 1, 3 * D)), full((2, 1, 3 * D)),
            full((D, ATOM_OUT)), full((1, ATOM_OUT)),
            full((D, BOND_OUT)), full((D, BOND_OUT)), full((1, BOND_OUT)),
        ],
        out_specs=pl.BlockSpec((1, L, ATOM_OUT + K * BOND_OUT),
                               lambda i: (i, 0, 0)),
    )
    return pl.pallas_call(
        _grn_kernel,
        grid_spec=grid_spec,
        out_shape=jax.ShapeDtypeStruct((B, L, ATOM_OUT + K * BOND_OUT),
                                       jnp.float32),
    )(idx, am, mf3, activated_features,
      wm1, wm2, bma, wmat, bmat, mwih, mwhh, mbih, mbhh,
      w1c, w2c, bal, wat, batt, gwih, gwhh, gbih, gbhh,
      wafc, bafc, wb1, wb2, bb)


# speedup vs baseline: 1.5960x; 1.5960x over previous
import jax
import jax.numpy as jnp
from jax.experimental import pallas as pl

B = 256
L = 96
K = 6
D = 128
ATOM_OUT = 40
BOND_OUT = 10
NEG = -9e8
MB = 4
R = MB * L


def _elu(x):
    return jnp.where(x > 0, x, jnp.exp(jnp.minimum(x, 0.0)) - 1.0)


def _gru_block(x, h, wih, whh, bih, bhh):
    gi = jnp.dot(x, wih, preferred_element_type=jnp.float32) + bih
    gh = jnp.dot(h, whh, preferred_element_type=jnp.float32) + bhh
    r = jax.nn.sigmoid(gi[:, :D] + gh[:, :D])
    z = jax.nn.sigmoid(gi[:, D:2 * D] + gh[:, D:2 * D])
    n = jnp.tanh(gi[:, 2 * D:] + r * gh[:, 2 * D:])
    return (1.0 - z) * n + z * h


def _seg_softmax(x, io, lo, hi):
    m = (io >= lo) & (io < hi)
    xs = jnp.where(m, x, NEG)
    mx = jnp.max(xs, axis=-1, keepdims=True)
    e = jnp.exp(xs - mx) * m.astype(jnp.float32)
    return e / jnp.sum(e, axis=-1, keepdims=True)


def _grn_kernel(idx_ref, am_ref, mf_ref, af_ref,
                wm1_ref, wm2_ref, bma_ref, wmat_ref, bmat_ref,
                mwih_ref, mwhh_ref, mbih_ref, mbhh_ref,
                w1c_ref, w2c_ref, bal_ref, wat_ref, batt_ref,
                gwih_ref, gwhh_ref, gbih_ref, gbhh_ref,
                wafc_ref, bafc_ref, wb1_ref, wb2_ref, bb_ref,
                out_ref):
    idx = idx_ref[...].reshape(R, K)
    am = am_ref[...].reshape(R, 1)
    mfm = mf_ref[...].reshape(MB, D)
    af = af_ref[...].reshape(R, D)

    rio = jax.lax.broadcasted_iota(jnp.int32, (R, MB), 0) // L
    cio = jax.lax.broadcasted_iota(jnp.int32, (R, MB), 1)
    sel = (rio == cio).astype(jnp.float32)

    mfh = jnp.dot(mfm, wm1_ref[...], preferred_element_type=jnp.float32)
    mfh_b = jnp.dot(sel, mfh, preferred_element_type=jnp.float32)
    mf_b = jnp.dot(sel, mfm, preferred_element_type=jnp.float32)
    afh = jnp.dot(af, wm2_ref[...], preferred_element_type=jnp.float32)
    v = jax.nn.leaky_relu(mfh_b + afh + bma_ref[...])
    msm = jnp.where(am == 0.0, NEG, 0.0)
    v = (v + msm) * am
    giT = mf_b * af
    ctx = _elu(
        jnp.dot(v * af, wmat_ref[...], preferred_element_type=jnp.float32)
        + bmat_ref[...])
    act = jax.nn.relu(_gru_block(ctx, giT, mwih_ref[...], mwhh_ref[...],
                                 mbih_ref[...], mbhh_ref[...]))

    jio = jax.lax.broadcasted_iota(jnp.int32, (L, L), 1)
    mk = [[(idx[m * L:(m + 1) * L, k:k + 1] == jio).astype(jnp.float32)
           for k in range(K)] for m in range(MB)]
    amask = jnp.where(idx != L - 1, 1.0, 0.0)
    smask = jnp.where(idx == L - 1, NEG, 0.0)

    for d in range(2):
        s_self = jnp.dot(act, w1c_ref[d], preferred_element_type=jnp.float32)
        p_col = jnp.dot(act, w2c_ref[d], preferred_element_type=jnp.float32)
        p_g = jnp.concatenate(
            [jnp.concatenate(
                [jnp.dot(mk[m][k], p_col[m * L:(m + 1) * L],
                         preferred_element_type=jnp.float32)
                 for k in range(K)], axis=-1)
             for m in range(MB)], axis=0)
        score = jax.nn.leaky_relu(s_self + p_g + bal_ref[d]) + smask
        mx = jnp.max(score, axis=1, keepdims=True)
        e = jnp.exp(score - mx)
        attn = e / jnp.sum(e, axis=1, keepdims=True) * amask
        ctxw_parts = []
        for m in range(MB):
            s_mat = attn[m * L:(m + 1) * L, 0:1] * mk[m][0]
            for k in range(1, K):
                s_mat = s_mat + attn[m * L:(m + 1) * L, k:k + 1] * mk[m][k]
            ctxw_parts.append(
                jnp.dot(s_mat, act[m * L:(m + 1) * L],
                        preferred_element_type=jnp.float32))
        ctxw = jnp.concatenate(ctxw_parts, axis=0)
        asum = jnp.sum(attn, axis=1, keepdims=True)
        ctx2 = _elu(
            jnp.dot(ctxw, wat_ref[d], preferred_element_type=jnp.float32)
            + asum * batt_ref[d])
        act = jax.nn.relu(_gru_block(ctx2, act, gwih_ref[d], gwhh_ref[d],
                                     gbih_ref[d], gbhh_ref[d]))

    atom_out = (jnp.dot(act, wafc_ref[...], preferred_element_type=jnp.float32)
                + bafc_ref[...])
    io40 = jax.lax.broadcasted_iota(jnp.int32, (R, ATOM_OUT), 1)
    a = (_seg_softmax(atom_out, io40, 0, 16)
         + _seg_softmax(atom_out, io40, 16, 22)
         + _seg_softmax(atom_out, io40, 24, 30)
         + _seg_softmax(atom_out, io40, 31, 36)
         + _seg_softmax(atom_out, io40, 37, 39))
    a = a + jnp.where(io40 == 24, jax.nn.relu(atom_out), 0.0)
    a = a + jnp.where(io40 == 30, jax.nn.sigmoid(atom_out), 0.0)
    a = a + jnp.where(io40 == 36, jax.nn.sigmoid(atom_out), 0.0)

    r_self = jnp.dot(act, wb1_ref[...], preferred_element_type=jnp.float32)
    q = jnp.dot(act, wb2_ref[...], preferred_element_type=jnp.float32)
    io10 = jax.lax.broadcasted_iota(jnp.int32, (R, BOND_OUT), 1)
    pieces = [a]
    for k in range(K):
        qg = jnp.concatenate(
            [jnp.dot(mk[m][k], q[m * L:(m + 1) * L],
                     preferred_element_type=jnp.float32)
             for m in range(MB)], axis=0)
        bo = r_self + qg + bb_ref[...]
        pieces.append(_seg_softmax(bo, io10, 0, 4)
                      + _seg_softmax(bo, io10, 6, 10))
    out = jnp.concatenate(pieces, axis=-1)
    out_ref[...] = out.reshape(MB, L, ATOM_OUT + K * BOND_OUT)


@jax.jit
def kernel(atom_list, bond_list, atom_degree_list, bond_degree_list, atom_mask,
           mol_feature, activated_features, W_atom_fc, b_atom_fc, W_bond_fc,
           b_bond_fc, gru_W_ih, gru_W_hh, gru_b_ih, gru_b_hh, W_align, b_align,
           W_attend, b_attend, mol_gru_W_ih, mol_gru_W_hh, mol_gru_b_ih,
           mol_gru_b_hh, W_mol_align, b_mol_align, W_mol_attend, b_mol_attend):
    del atom_list, bond_list, bond_degree_list

    idx = atom_degree_list.astype(jnp.int32)
    am = atom_mask.reshape(B, L, 1)
    mf3 = mol_feature.reshape(B, 1, D)

    wm1 = W_mol_align[:, :D].T
    wm2 = W_mol_align[:, D:].T
    bma = b_mol_align[None, :]
    wmat = W_mol_attend.T
    bmat = b_mol_attend[None, :]
    mwih = mol_gru_W_ih.T
    mwhh = mol_gru_W_hh.T
    mbih = mol_gru_b_ih[None, :]
    mbhh = mol_gru_b_hh[None, :]
    w1c = jnp.stack([W_align[0, :, :D].T, W_align[1, :, :D].T])
    w2c = jnp.stack([W_align[0, :, D:].T, W_align[1, :, D:].T])
    bal = b_align[:2].reshape(2, 1, 1)
    wat = jnp.stack([W_attend[0].T, W_attend[1].T])
    batt = b_attend[:2].reshape(2, 1, D)
    gwih = jnp.stack([gru_W_ih[0].T, gru_W_ih[1].T])
    gwhh = jnp.stack([gru_W_hh[0].T, gru_W_hh[1].T])
    gbih = gru_b_ih[:2].reshape(2, 1, 3 * D)
    gbhh = gru_b_hh[:2].reshape(2, 1, 3 * D)
    wafc = W_atom_fc.T
    bafc = b_atom_fc[None, :]
    wb1 = W_bond_fc[:, :D].T
    wb2 = W_bond_fc[:, D:].T
    bb = b_bond_fc[None, :]

    full = lambda shape: pl.BlockSpec(shape, lambda i: (0,) * len(shape))
    grid_spec = pl.GridSpec(
        grid=(B // MB,),
        in_specs=[
            pl.BlockSpec((MB, L, K), lambda i: (i, 0, 0)),
            pl.BlockSpec((MB, L, 1), lambda i: (i, 0, 0)),
            pl.BlockSpec((MB, 1, D), lambda i: (i, 0, 0)),
            pl.BlockSpec((MB, L, D), lambda i: (i, 0, 0)),
            full((D, D)), full((D, D)), full((1, D)), full((D, D)),
            full((1, D)), full((D, 3 * D)), full((D, 3 * D)),
            full((1, 3 * D)), full((1, 3 * D)),
            full((2, D, 1)), full((2, D, 1)), full((2, 1, 1)),
            full((2, D, D)), full((2, 1, D)),
            full((2, D, 3 * D)), full((2, D, 3 * D)),
            full((2, 1, 3 * D)), full((2, 1, 3 * D)),
            full((D, ATOM_OUT)), full((1, ATOM_OUT)),
            full((D, BOND_OUT)), full((D, BOND_OUT)), full((1, BOND_OUT)),
        ],
        out_specs=pl.BlockSpec((MB, L, ATOM_OUT + K * BOND_OUT),
                               lambda i: (i, 0, 0)),
    )
    return pl.pallas_call(
        _grn_kernel,
        grid_spec=grid_spec,
        out_shape=jax.ShapeDtypeStruct((B, L, ATOM_OUT + K * BOND_OUT),
                                       jnp.float32),
    )(idx, am, mf3, activated_features,
      wm1, wm2, bma, wmat, bmat, mwih, mwhh, mbih, mbhh,
      w1c, w2c, bal, wat, batt, gwih, gwhh, gbih, gbhh,
      wafc, bafc, wb1, wb2, bb)


# speedup vs baseline: 2.2623x; 1.4174x over previous
import jax
import jax.numpy as jnp
from jax.experimental import pallas as pl

B = 256
L = 96
K = 6
D = 128
ATOM_OUT = 40
BOND_OUT = 10
NEG = -9e8
MB = 8
R = MB * L


def _elu(x):
    return jnp.where(x > 0, x, jnp.exp(jnp.minimum(x, 0.0)) - 1.0)


def _gru_block(x, h, wih, whh, bih, bhh):
    gi = jnp.dot(x, wih, preferred_element_type=jnp.float32) + bih
    gh = jnp.dot(h, whh, preferred_element_type=jnp.float32) + bhh
    r = jax.nn.sigmoid(gi[:, :D] + gh[:, :D])
    z = jax.nn.sigmoid(gi[:, D:2 * D] + gh[:, D:2 * D])
    n = jnp.tanh(gi[:, 2 * D:] + r * gh[:, 2 * D:])
    return (1.0 - z) * n + z * h


def _seg_softmax(x, io, lo, hi):
    m = (io >= lo) & (io < hi)
    xs = jnp.where(m, x, NEG)
    mx = jnp.max(xs, axis=-1, keepdims=True)
    e = jnp.exp(xs - mx) * m.astype(jnp.float32)
    return e / jnp.sum(e, axis=-1, keepdims=True)


def _grn_kernel(idx_ref, am_ref, mf_ref, af_ref,
                wm1_ref, wm2_ref, bma_ref, wmat_ref, bmat_ref,
                mwih_ref, mwhh_ref, mbih_ref, mbhh_ref,
                w1c_ref, w2c_ref, bal_ref, wat_ref, batt_ref,
                gwih_ref, gwhh_ref, gbih_ref, gbhh_ref,
                wafc_ref, bafc_ref, wb1_ref, wb2_ref, bb_ref,
                out_ref):
    idx = idx_ref[...].reshape(R, K)
    am = am_ref[...].reshape(R, 1)
    mfm = mf_ref[...].reshape(MB, D)
    af = af_ref[...].reshape(R, D)

    rio = jax.lax.broadcasted_iota(jnp.int32, (R, MB), 0) // L
    cio = jax.lax.broadcasted_iota(jnp.int32, (R, MB), 1)
    sel = (rio == cio).astype(jnp.float32)

    mfh = jnp.dot(mfm, wm1_ref[...], preferred_element_type=jnp.float32)
    mfh_b = jnp.dot(sel, mfh, preferred_element_type=jnp.float32)
    mf_b = jnp.dot(sel, mfm, preferred_element_type=jnp.float32)
    afh = jnp.dot(af, wm2_ref[...], preferred_element_type=jnp.float32)
    v = jax.nn.leaky_relu(mfh_b + afh + bma_ref[...])
    msm = jnp.where(am == 0.0, NEG, 0.0)
    v = (v + msm) * am
    giT = mf_b * af
    ctx = _elu(
        jnp.dot(v * af, wmat_ref[...], preferred_element_type=jnp.float32)
        + bmat_ref[...])
    act = jax.nn.relu(_gru_block(ctx, giT, mwih_ref[...], mwhh_ref[...],
                                 mbih_ref[...], mbhh_ref[...]))

    jio = jax.lax.broadcasted_iota(jnp.int32, (L, L), 1)
    mk = [[(idx[m * L:(m + 1) * L, k:k + 1] == jio).astype(jnp.float32)
           for k in range(K)] for m in range(MB)]
    amask = jnp.where(idx != L - 1, 1.0, 0.0)
    smask = jnp.where(idx == L - 1, NEG, 0.0)

    for d in range(2):
        s_self = jnp.dot(act, w1c_ref[d], preferred_element_type=jnp.float32)
        p_col = jnp.dot(act, w2c_ref[d], preferred_element_type=jnp.float32)
        p_g = jnp.concatenate(
            [jnp.concatenate(
                [jnp.dot(mk[m][k], p_col[m * L:(m + 1) * L],
                         preferred_element_type=jnp.float32)
                 for k in range(K)], axis=-1)
             for m in range(MB)], axis=0)
        score = jax.nn.leaky_relu(s_self + p_g + bal_ref[d]) + smask
        mx = jnp.max(score, axis=1, keepdims=True)
        e = jnp.exp(score - mx)
        attn = e / jnp.sum(e, axis=1, keepdims=True) * amask
        ctxw_parts = []
        for m in range(MB):
            s_mat = attn[m * L:(m + 1) * L, 0:1] * mk[m][0]
            for k in range(1, K):
                s_mat = s_mat + attn[m * L:(m + 1) * L, k:k + 1] * mk[m][k]
            ctxw_parts.append(
                jnp.dot(s_mat, act[m * L:(m + 1) * L],
                        preferred_element_type=jnp.float32))
        ctxw = jnp.concatenate(ctxw_parts, axis=0)
        asum = jnp.sum(attn, axis=1, keepdims=True)
        ctx2 = _elu(
            jnp.dot(ctxw, wat_ref[d], preferred_element_type=jnp.float32)
            + asum * batt_ref[d])
        act = jax.nn.relu(_gru_block(ctx2, act, gwih_ref[d], gwhh_ref[d],
                                     gbih_ref[d], gbhh_ref[d]))

    atom_out = (jnp.dot(act, wafc_ref[...], preferred_element_type=jnp.float32)
                + bafc_ref[...])
    io40 = jax.lax.broadcasted_iota(jnp.int32, (R, ATOM_OUT), 1)
    a = (_seg_softmax(atom_out, io40, 0, 16)
         + _seg_softmax(atom_out, io40, 16, 22)
         + _seg_softmax(atom_out, io40, 24, 30)
         + _seg_softmax(atom_out, io40, 31, 36)
         + _seg_softmax(atom_out, io40, 37, 39))
    a = a + jnp.where(io40 == 24, jax.nn.relu(atom_out), 0.0)
    a = a + jnp.where(io40 == 30, jax.nn.sigmoid(atom_out), 0.0)
    a = a + jnp.where(io40 == 36, jax.nn.sigmoid(atom_out), 0.0)

    r_self = jnp.dot(act, wb1_ref[...], preferred_element_type=jnp.float32)
    q = jnp.dot(act, wb2_ref[...], preferred_element_type=jnp.float32)
    io10 = jax.lax.broadcasted_iota(jnp.int32, (R, BOND_OUT), 1)
    pieces = [a]
    for k in range(K):
        qg = jnp.concatenate(
            [jnp.dot(mk[m][k], q[m * L:(m + 1) * L],
                     preferred_element_type=jnp.float32)
             for m in range(MB)], axis=0)
        bo = r_self + qg + bb_ref[...]
        pieces.append(_seg_softmax(bo, io10, 0, 4)
                      + _seg_softmax(bo, io10, 6, 10))
    out = jnp.concatenate(pieces, axis=-1)
    out_ref[...] = out.reshape(MB, L, ATOM_OUT + K * BOND_OUT)


@jax.jit
def kernel(atom_list, bond_list, atom_degree_list, bond_degree_list, atom_mask,
           mol_feature, activated_features, W_atom_fc, b_atom_fc, W_bond_fc,
           b_bond_fc, gru_W_ih, gru_W_hh, gru_b_ih, gru_b_hh, W_align, b_align,
           W_attend, b_attend, mol_gru_W_ih, mol_gru_W_hh, mol_gru_b_ih,
           mol_gru_b_hh, W_mol_align, b_mol_align, W_mol_attend, b_mol_attend):
    del atom_list, bond_list, bond_degree_list

    idx = atom_degree_list.astype(jnp.int32)
    am = atom_mask.reshape(B, L, 1)
    mf3 = mol_feature.reshape(B, 1, D)

    wm1 = W_mol_align[:, :D].T
    wm2 = W_mol_align[:, D:].T
    bma = b_mol_align[None, :]
    wmat = W_mol_attend.T
    bmat = b_mol_attend[None, :]
    mwih = mol_gru_W_ih.T
    mwhh = mol_gru_W_hh.T
    mbih = mol_gru_b_ih[None, :]
    mbhh = mol_gru_b_hh[None, :]
    w1c = jnp.stack([W_align[0, :, :D].T, W_align[1, :, :D].T])
    w2c = jnp.stack([W_align[0, :, D:].T, W_align[1, :, D:].T])
    bal = b_align[:2].reshape(2, 1, 1)
    wat = jnp.stack([W_attend[0].T, W_attend[1].T])
    batt = b_attend[:2].reshape(2, 1, D)
    gwih = jnp.stack([gru_W_ih[0].T, gru_W_ih[1].T])
    gwhh = jnp.stack([gru_W_hh[0].T, gru_W_hh[1].T])
    gbih = gru_b_ih[:2].reshape(2, 1, 3 * D)
    gbhh = gru_b_hh[:2].reshape(2, 1, 3 * D)
    wafc = W_atom_fc.T
    bafc = b_atom_fc[None, :]
    wb1 = W_bond_fc[:, :D].T
    wb2 = W_bond_fc[:, D:].T
    bb = b_bond_fc[None, :]

    full = lambda shape: pl.BlockSpec(shape, lambda i: (0,) * len(shape))
    grid_spec = pl.GridSpec(
        grid=(B // MB,),
        in_specs=[
            pl.BlockSpec((MB, L, K), lambda i: (i, 0, 0)),
            pl.BlockSpec((MB, L, 1), lambda i: (i, 0, 0)),
            pl.BlockSpec((MB, 1, D), lambda i: (i, 0, 0)),
            pl.BlockSpec((MB, L, D), lambda i: (i, 0, 0)),
            full((D, D)), full((D, D)), full((1, D)), full((D, D)),
            full((1, D)), full((D, 3 * D)), full((D, 3 * D)),
            full((1, 3 * D)), full((1, 3 * D)),
            full((2, D, 1)), full((2, D, 1)), full((2, 1, 1)),
            full((2, D, D)), full((2, 1, D)),
            full((2, D, 3 * D)), full((2, D, 3 * D)),
            full((2, 1, 3 * D)), full((2, 1, 3 * D)),
            full((D, ATOM_OUT)), full((1, ATOM_OUT)),
            full((D, BOND_OUT)), full((D, BOND_OUT)), full((1, BOND_OUT)),
        ],
        out_specs=pl.BlockSpec((MB, L, ATOM_OUT + K * BOND_OUT),
                               lambda i: (i, 0, 0)),
    )
    return pl.pallas_call(
        _grn_kernel,
        grid_spec=grid_spec,
        out_shape=jax.ShapeDtypeStruct((B, L, ATOM_OUT + K * BOND_OUT),
                                       jnp.float32),
    )(idx, am, mf3, activated_features,
      wm1, wm2, bma, wmat, bmat, mwih, mwhh, mbih, mbhh,
      w1c, w2c, bal, wat, batt, gwih, gwhh, gbih, gbhh,
      wafc, bafc, wb1, wb2, bb)


# speedup vs baseline: 2.3647x; 1.0453x over previous
import jax
import jax.numpy as jnp
from jax.experimental import pallas as pl

B = 256
L = 96
K = 6
D = 128
ATOM_OUT = 40
BOND_OUT = 10
NEG = -9e8
MB = 16
R = MB * L


def _elu(x):
    return jnp.where(x > 0, x, jnp.exp(jnp.minimum(x, 0.0)) - 1.0)


def _gru_block(x, h, wih, whh, bih, bhh):
    gi = jnp.dot(x, wih, preferred_element_type=jnp.float32) + bih
    gh = jnp.dot(h, whh, preferred_element_type=jnp.float32) + bhh
    r = jax.nn.sigmoid(gi[:, :D] + gh[:, :D])
    z = jax.nn.sigmoid(gi[:, D:2 * D] + gh[:, D:2 * D])
    n = jnp.tanh(gi[:, 2 * D:] + r * gh[:, 2 * D:])
    return (1.0 - z) * n + z * h


def _seg_softmax(x, io, lo, hi):
    m = (io >= lo) & (io < hi)
    xs = jnp.where(m, x, NEG)
    mx = jnp.max(xs, axis=-1, keepdims=True)
    e = jnp.exp(xs - mx) * m.astype(jnp.float32)
    return e / jnp.sum(e, axis=-1, keepdims=True)


def _grn_kernel(idx_ref, am_ref, mf_ref, af_ref,
                wm1_ref, wm2_ref, bma_ref, wmat_ref, bmat_ref,
                mwih_ref, mwhh_ref, mbih_ref, mbhh_ref,
                w1c_ref, w2c_ref, bal_ref, wat_ref, batt_ref,
                gwih_ref, gwhh_ref, gbih_ref, gbhh_ref,
                wafc_ref, bafc_ref, wb1_ref, wb2_ref, bb_ref,
                out_ref):
    idx = idx_ref[...].reshape(R, K)
    am = am_ref[...].reshape(R, 1)
    mfm = mf_ref[...].reshape(MB, D)
    af = af_ref[...].reshape(R, D)

    rio = jax.lax.broadcasted_iota(jnp.int32, (R, MB), 0) // L
    cio = jax.lax.broadcasted_iota(jnp.int32, (R, MB), 1)
    sel = (rio == cio).astype(jnp.float32)

    mfh = jnp.dot(mfm, wm1_ref[...], preferred_element_type=jnp.float32)
    mfh_b = jnp.dot(sel, mfh, preferred_element_type=jnp.float32)
    mf_b = jnp.dot(sel, mfm, preferred_element_type=jnp.float32)
    afh = jnp.dot(af, wm2_ref[...], preferred_element_type=jnp.float32)
    v = jax.nn.leaky_relu(mfh_b + afh + bma_ref[...])
    msm = jnp.where(am == 0.0, NEG, 0.0)
    v = (v + msm) * am
    giT = mf_b * af
    ctx = _elu(
        jnp.dot(v * af, wmat_ref[...], preferred_element_type=jnp.float32)
        + bmat_ref[...])
    act = jax.nn.relu(_gru_block(ctx, giT, mwih_ref[...], mwhh_ref[...],
                                 mbih_ref[...], mbhh_ref[...]))

    jio = jax.lax.broadcasted_iota(jnp.int32, (L, L), 1)
    mk = [[(idx[m * L:(m + 1) * L, k:k + 1] == jio).astype(jnp.float32)
           for k in range(K)] for m in range(MB)]
    amask = jnp.where(idx != L - 1, 1.0, 0.0)
    smask = jnp.where(idx == L - 1, NEG, 0.0)

    for d in range(2):
        s_self = jnp.dot(act, w1c_ref[d], preferred_element_type=jnp.float32)
        p_col = jnp.dot(act, w2c_ref[d], preferred_element_type=jnp.float32)
        p_g = jnp.concatenate(
            [jnp.concatenate(
                [jnp.dot(mk[m][k], p_col[m * L:(m + 1) * L],
                         preferred_element_type=jnp.float32)
                 for k in range(K)], axis=-1)
             for m in range(MB)], axis=0)
        score = jax.nn.leaky_relu(s_self + p_g + bal_ref[d]) + smask
        mx = jnp.max(score, axis=1, keepdims=True)
        e = jnp.exp(score - mx)
        attn = e / jnp.sum(e, axis=1, keepdims=True) * amask
        ctxw_parts = []
        for m in range(MB):
            s_mat = attn[m * L:(m + 1) * L, 0:1] * mk[m][0]
            for k in range(1, K):
                s_mat = s_mat + attn[m * L:(m + 1) * L, k:k + 1] * mk[m][k]
            ctxw_parts.append(
                jnp.dot(s_mat, act[m * L:(m + 1) * L],
                        preferred_element_type=jnp.float32))
        ctxw = jnp.concatenate(ctxw_parts, axis=0)
        asum = jnp.sum(attn, axis=1, keepdims=True)
        ctx2 = _elu(
            jnp.dot(ctxw, wat_ref[d], preferred_element_type=jnp.float32)
            + asum * batt_ref[d])
        act = jax.nn.relu(_gru_block(ctx2, act, gwih_ref[d], gwhh_ref[d],
                                     gbih_ref[d], gbhh_ref[d]))

    atom_out = (jnp.dot(act, wafc_ref[...], preferred_element_type=jnp.float32)
                + bafc_ref[...])
    io40 = jax.lax.broadcasted_iota(jnp.int32, (R, ATOM_OUT), 1)
    a = (_seg_softmax(atom_out, io40, 0, 16)
         + _seg_softmax(atom_out, io40, 16, 22)
         + _seg_softmax(atom_out, io40, 24, 30)
         + _seg_softmax(atom_out, io40, 31, 36)
         + _seg_softmax(atom_out, io40, 37, 39))
    a = a + jnp.where(io40 == 24, jax.nn.relu(atom_out), 0.0)
    a = a + jnp.where(io40 == 30, jax.nn.sigmoid(atom_out), 0.0)
    a = a + jnp.where(io40 == 36, jax.nn.sigmoid(atom_out), 0.0)

    r_self = jnp.dot(act, wb1_ref[...], preferred_element_type=jnp.float32)
    q = jnp.dot(act, wb2_ref[...], preferred_element_type=jnp.float32)
    io10 = jax.lax.broadcasted_iota(jnp.int32, (R, BOND_OUT), 1)
    pieces = [a]
    for k in range(K):
        qg = jnp.concatenate(
            [jnp.dot(mk[m][k], q[m * L:(m + 1) * L],
                     preferred_element_type=jnp.float32)
             for m in range(MB)], axis=0)
        bo = r_self + qg + bb_ref[...]
        pieces.append(_seg_softmax(bo, io10, 0, 4)
                      + _seg_softmax(bo, io10, 6, 10))
    out = jnp.concatenate(pieces, axis=-1)
    out_ref[...] = out.reshape(MB, L, ATOM_OUT + K * BOND_OUT)


@jax.jit
def kernel(atom_list, bond_list, atom_degree_list, bond_degree_list, atom_mask,
           mol_feature, activated_features, W_atom_fc, b_atom_fc, W_bond_fc,
           b_bond_fc, gru_W_ih, gru_W_hh, gru_b_ih, gru_b_hh, W_align, b_align,
           W_attend, b_attend, mol_gru_W_ih, mol_gru_W_hh, mol_gru_b_ih,
           mol_gru_b_hh, W_mol_align, b_mol_align, W_mol_attend, b_mol_attend):
    del atom_list, bond_list, bond_degree_list

    idx = atom_degree_list.astype(jnp.int32)
    am = atom_mask.reshape(B, L, 1)
    mf3 = mol_feature.reshape(B, 1, D)

    wm1 = W_mol_align[:, :D].T
    wm2 = W_mol_align[:, D:].T
    bma = b_mol_align[None, :]
    wmat = W_mol_attend.T
    bmat = b_mol_attend[None, :]
    mwih = mol_gru_W_ih.T
    mwhh = mol_gru_W_hh.T
    mbih = mol_gru_b_ih[None, :]
    mbhh = mol_gru_b_hh[None, :]
    w1c = jnp.stack([W_align[0, :, :D].T, W_align[1, :, :D].T])
    w2c = jnp.stack([W_align[0, :, D:].T, W_align[1, :, D:].T])
    bal = b_align[:2].reshape(2, 1, 1)
    wat = jnp.stack([W_attend[0].T, W_attend[1].T])
    batt = b_attend[:2].reshape(2, 1, D)
    gwih = jnp.stack([gru_W_ih[0].T, gru_W_ih[1].T])
    gwhh = jnp.stack([gru_W_hh[0].T, gru_W_hh[1].T])
    gbih = gru_b_ih[:2].reshape(2, 1, 3 * D)
    gbhh = gru_b_hh[:2].reshape(2, 1, 3 * D)
    wafc = W_atom_fc.T
    bafc = b_atom_fc[None, :]
    wb1 = W_bond_fc[:, :D].T
    wb2 = W_bond_fc[:, D:].T
    bb = b_bond_fc[None, :]

    full = lambda shape: pl.BlockSpec(shape, lambda i: (0,) * len(shape))
    grid_spec = pl.GridSpec(
        grid=(B // MB,),
        in_specs=[
            pl.BlockSpec((MB, L, K), lambda i: (i, 0, 0)),
            pl.BlockSpec((MB, L, 1), lambda i: (i, 0, 0)),
            pl.BlockSpec((MB, 1, D), lambda i: (i, 0, 0)),
            pl.BlockSpec((MB, L, D), lambda i: (i, 0, 0)),
            full((D, D)), full((D, D)), full((1, D)), full((D, D)),
            full((1, D)), full((D, 3 * D)), full((D, 3 * D)),
            full((1, 3 * D)), full((1, 3 * D)),
            full((2, D, 1)), full((2, D, 1)), full((2, 1, 1)),
            full((2, D, D)), full((2, 1, D)),
            full((2, D, 3 * D)), full((2, D, 3 * D)),
            full((2, 1, 3 * D)), full((2, 1, 3 * D)),
            full((D, ATOM_OUT)), full((1, ATOM_OUT)),
            full((D, BOND_OUT)), full((D, BOND_OUT)), full((1, BOND_OUT)),
        ],
        out_specs=pl.BlockSpec((MB, L, ATOM_OUT + K * BOND_OUT),
                               lambda i: (i, 0, 0)),
    )
    return pl.pallas_call(
        _grn_kernel,
        grid_spec=grid_spec,
        out_shape=jax.ShapeDtypeStruct((B, L, ATOM_OUT + K * BOND_OUT),
                                       jnp.float32),
    )(idx, am, mf3, activated_features,
      wm1, wm2, bma, wmat, bmat, mwih, mwhh, mbih, mbhh,
      w1c, w2c, bal, wat, batt, gwih, gwhh, gbih, gbhh,
      wafc, bafc, wb1, wb2, bb)
